# baseline (speedup 1.0000x reference)
import jax
import jax.numpy as jnp
from jax.experimental import pallas as pl
from jax.experimental.pallas import tpu as pltpu

B = 100
N = 1000
E = 128

_B_CHUNKS = 10
_BC = B // _B_CHUNKS


def _body(cur_sm, feat_ref, act_ref, stc_ref, w5_ref, b5_ref, w6_ref,
          b6_ref, w7_ref, b7_ref, w8_ref, b8_ref, w9_ref, b9_ref,
          q_ref, acc_ref, curacc_ref):
    i = pl.program_id(0)

    s = jnp.sum(feat_ref[...], axis=1)
    for k in range(_B_CHUNKS):
        @pl.when(i == k)
        def _():
            acc_ref[k * _BC:(k + 1) * _BC] = s
            for g in range(_BC):
                curacc_ref[k * _BC + g] = feat_ref[g, cur_sm[k * _BC + g]]

    @pl.when(i == _B_CHUNKS - 1)
    def _():
        feat_sum = acc_ref[...]
        cur_feat = curacc_ref[...]
        act = jnp.transpose(act_ref[...])
        stc = jnp.transpose(stc_ref[...])
        w5a = w5_ref[...][:E]
        w5b = w5_ref[...][E:]
        b6 = jnp.reshape(b6_ref[...], (1, E))
        b7 = jnp.reshape(b7_ref[...], (1, E))
        b8 = jnp.reshape(b8_ref[...], (1, E))
        b9 = jnp.reshape(b9_ref[...], (1, E))
        w8 = jnp.reshape(w8_ref[...], (1, E))
        w9 = jnp.reshape(w9_ref[...], (1, E))
        h1 = jnp.dot(feat_sum, w6_ref[...], precision=jax.lax.Precision.HIGHEST,
                     preferred_element_type=jnp.float32) + b6
        h2 = (jnp.dot(cur_feat, w7_ref[...], precision=jax.lax.Precision.HIGHEST,
                      preferred_element_type=jnp.float32) + b7
              + act * w8 + b8 + stc * w9 + b9)
        q = jnp.sum(jnp.maximum(h1, 0.0) * w5a[None, :]
                    + jnp.maximum(h2, 0.0) * w5b[None, :],
                    axis=1, keepdims=True) + b5_ref[0]
        q_ref[...] = jnp.transpose(q)


def kernel(feat, cur_node, action, state_c, W5, b5, W6, b6, W7, b7, W8, b8,
           W9, b9):
    full = lambda shape: pl.BlockSpec(shape, lambda i, *_: (0,) * len(shape))
    return pl.pallas_call(
        _body,
        grid_spec=pltpu.PrefetchScalarGridSpec(
            num_scalar_prefetch=1,
            grid=(_B_CHUNKS,),
            in_specs=[
                pl.BlockSpec((_BC, N, E), lambda i, *_: (i, 0, 0)),
                full((1, B)),
                full((1, B)),
                full((2 * E,)),
                full((1,)),
                full((E, E)),
                full((E,)),
                full((E, E)),
                full((E,)),
                full((1, E)),
                full((E,)),
                full((1, E)),
                full((E,)),
            ],
            out_specs=pl.BlockSpec((1, B), lambda i, *_: (0, 0)),
            scratch_shapes=[pltpu.VMEM((B, E), jnp.float32),
                            pltpu.VMEM((B, E), jnp.float32)],
        ),
        out_shape=jax.ShapeDtypeStruct((1, B), jnp.float32),
    )(cur_node.astype(jnp.int32), feat, action.reshape(1, B),
      state_c.reshape(1, B), W5.reshape(2 * E), b5, W6, b6, W7, b7, W8, b8,
      W9, b9).reshape(B, 1)


# speedup vs baseline: 1.0116x; 1.0116x over previous
import jax
import jax.numpy as jnp
from jax.experimental import pallas as pl
from jax.experimental.pallas import tpu as pltpu

B = 100
N = 1000
E = 128

_B_CHUNKS = 10
_BC = B // _B_CHUNKS


def _body(cur_sm, feat_ref, act_ref, stc_ref, w5_ref, b5_ref, w6_ref,
          b6_ref, w7_ref, b7_ref, w8_ref, b8_ref, w9_ref, b9_ref,
          q_ref, acc_ref, curacc_ref):
    i = pl.program_id(0)

    s = jnp.sum(feat_ref[...], axis=1)
    for k in range(_B_CHUNKS):
        @pl.when(i == k)
        def _():
            acc_ref[k * _BC:(k + 1) * _BC] = s
            for g in range(_BC):
                curacc_ref[k * _BC + g] = feat_ref[g, cur_sm[k * _BC + g]]

    @pl.when(i == _B_CHUNKS - 1)
    def _():
        feat_sum = acc_ref[...]
        cur_feat = curacc_ref[...]
        act = jnp.transpose(act_ref[...])
        stc = jnp.transpose(stc_ref[...])
        w5a = w5_ref[...][:E]
        w5b = w5_ref[...][E:]
        b6 = jnp.reshape(b6_ref[...], (1, E))
        b7 = jnp.reshape(b7_ref[...], (1, E))
        b8 = jnp.reshape(b8_ref[...], (1, E))
        b9 = jnp.reshape(b9_ref[...], (1, E))
        w8 = jnp.reshape(w8_ref[...], (1, E))
        w9 = jnp.reshape(w9_ref[...], (1, E))
        h1 = jnp.dot(feat_sum, w6_ref[...],
                     preferred_element_type=jnp.float32) + b6
        h2 = (jnp.dot(cur_feat, w7_ref[...],
                      preferred_element_type=jnp.float32) + b7
              + act * w8 + b8 + stc * w9 + b9)
        q = jnp.sum(jnp.maximum(h1, 0.0) * w5a[None, :]
                    + jnp.maximum(h2, 0.0) * w5b[None, :],
                    axis=1, keepdims=True) + b5_ref[0]
        q_ref[...] = jnp.transpose(q)


def kernel(feat, cur_node, action, state_c, W5, b5, W6, b6, W7, b7, W8, b8,
           W9, b9):
    full = lambda shape: pl.BlockSpec(shape, lambda i, *_: (0,) * len(shape))
    return pl.pallas_call(
        _body,
        grid_spec=pltpu.PrefetchScalarGridSpec(
            num_scalar_prefetch=1,
            grid=(_B_CHUNKS,),
            in_specs=[
                pl.BlockSpec((_BC, N, E), lambda i, *_: (i, 0, 0)),
                full((1, B)),
                full((1, B)),
                full((2 * E,)),
                full((1,)),
                full((E, E)),
                full((E,)),
                full((E, E)),
                full((E,)),
                full((1, E)),
                full((E,)),
                full((1, E)),
                full((E,)),
            ],
            out_specs=pl.BlockSpec((1, B), lambda i, *_: (0, 0)),
            scratch_shapes=[pltpu.VMEM((B, E), jnp.float32),
                            pltpu.VMEM((B, E), jnp.float32)],
        ),
        out_shape=jax.ShapeDtypeStruct((1, B), jnp.float32),
    )(cur_node.astype(jnp.int32), feat, action.reshape(1, B),
      state_c.reshape(1, B), W5.reshape(2 * E), b5, W6, b6, W7, b7, W8, b8,
      W9, b9).reshape(B, 1)
